# baseline (speedup 1.0000x reference)
import functools

import jax
import jax.numpy as jnp
from jax import lax
from jax.experimental import pallas as pl
from jax.experimental.pallas import tpu as pltpu
from jax.experimental.pallas import tpu_sc as plsc

EMB_DIM = 64
_DPAD = 128
BATCH = 4096
SIG_LEN = 50
_LPAD = 56
NUM_IDX = BATCH * SIG_LEN

_NC = 2
_NS = 16
_NW = _NC * _NS
_ROWS_PER_W = BATCH // _NW
_B_PER_W = NUM_IDX // _NW
_CROWS = 16
_CHUNK = _CROWS * SIG_LEN
_NCHUNK = _ROWS_PER_W // _CROWS
_NBUF = 2


def _make_gather():
    mesh = plsc.VectorSubcoreMesh(core_axis_name="c", subcore_axis_name="s")

    @functools.partial(
        pl.kernel,
        mesh=mesh,
        out_type=jax.ShapeDtypeStruct((BATCH, _LPAD, _DPAD), jnp.float32),
        scratch_types=[
            pltpu.VMEM((_B_PER_W,), jnp.int32),
            pltpu.VMEM((_NBUF, _CHUNK, EMB_DIM), jnp.float32),
            pltpu.SemaphoreType.DMA((_NBUF,)),
            pltpu.SemaphoreType.DMA((_NBUF,)),
        ],
        compiler_params=pltpu.CompilerParams(use_tc_tiling_on_sc=False),
    )
    def gather_kernel(idx_hbm, table_hbm, out_hbm, idx_v, rows_v, gsem, osem):
        wid = lax.axis_index("s") * _NC + lax.axis_index("c")
        ibase = pl.multiple_of(wid * _B_PER_W, _B_PER_W)
        bbase = wid * _ROWS_PER_W
        pltpu.sync_copy(idx_hbm.at[pl.ds(ibase, _B_PER_W)], idx_v)

        def gather(j):
            b = j % _NBUF
            return pltpu.async_copy(
                table_hbm.at[idx_v.at[pl.ds(j * _CHUNK, _CHUNK)]],
                rows_v.at[b],
                gsem.at[b],
            )

        def writeback(j):
            b = j % _NBUF
            cps = []
            for r in range(_CROWS):
                cps.append(
                    pltpu.async_copy(
                        rows_v.at[b].at[pl.ds(r * SIG_LEN, SIG_LEN)],
                        out_hbm.at[bbase + j * _CROWS + r].at[
                            pl.ds(0, SIG_LEN), pl.ds(0, EMB_DIM)
                        ],
                        osem.at[b],
                    )
                )
            return cps

        gathers = [gather(j) for j in range(_NBUF - 1)]
        writebacks = [None] * _NCHUNK
        for j in range(_NCHUNK):
            gathers[j].wait()
            writebacks[j] = writeback(j)
            k = j + _NBUF - 1
            if k < _NCHUNK:
                if k >= _NBUF:
                    for cp in writebacks[k - _NBUF]:
                        cp.wait()
                gathers.append(gather(k))
        for j in range(_NCHUNK - _NBUF, _NCHUNK):
            for cp in writebacks[j]:
                cp.wait()

    return gather_kernel


_gather = _make_gather()


@jax.jit
def kernel(news_batch, table):
    idx2 = news_batch.astype(jnp.int32).reshape(NUM_IDX) * 2
    table_p = jnp.pad(table, ((0, 0), (0, _DPAD - EMB_DIM)))
    table_2m = table_p.reshape(2 * 1000000, EMB_DIM)
    out_p = _gather(idx2, table_2m)
    return out_p[:, :SIG_LEN, :EMB_DIM]


# speedup vs baseline: 1.0064x; 1.0064x over previous
import functools

import jax
import jax.numpy as jnp
from jax import lax
from jax.experimental import pallas as pl
from jax.experimental.pallas import tpu as pltpu
from jax.experimental.pallas import tpu_sc as plsc

EMB_DIM = 64
_DPAD = 128
BATCH = 4096
SIG_LEN = 50
_LPAD = 56
NUM_IDX = BATCH * SIG_LEN

_NC = 2
_NS = 16
_NW = _NC * _NS
_ROWS_PER_W = BATCH // _NW
_B_PER_W = NUM_IDX // _NW
_CROWS = 8
_CHUNK = _CROWS * SIG_LEN
_NCHUNK = _ROWS_PER_W // _CROWS
_NBUF = 4


def _make_gather():
    mesh = plsc.VectorSubcoreMesh(core_axis_name="c", subcore_axis_name="s")

    @functools.partial(
        pl.kernel,
        mesh=mesh,
        out_type=jax.ShapeDtypeStruct((BATCH, _LPAD, _DPAD), jnp.float32),
        scratch_types=[
            pltpu.VMEM((_B_PER_W,), jnp.int32),
            pltpu.VMEM((_NBUF, _CHUNK, EMB_DIM), jnp.float32),
            pltpu.SemaphoreType.DMA((_NBUF,)),
            pltpu.SemaphoreType.DMA((_NBUF,)),
        ],
        compiler_params=pltpu.CompilerParams(use_tc_tiling_on_sc=False),
    )
    def gather_kernel(idx_hbm, table_hbm, out_hbm, idx_v, rows_v, gsem, osem):
        wid = lax.axis_index("s") * _NC + lax.axis_index("c")
        ibase = pl.multiple_of(wid * _B_PER_W, _B_PER_W)
        bbase = wid * _ROWS_PER_W
        pltpu.sync_copy(idx_hbm.at[pl.ds(ibase, _B_PER_W)], idx_v)

        def gather(j):
            b = j % _NBUF
            return pltpu.async_copy(
                table_hbm.at[idx_v.at[pl.ds(j * _CHUNK, _CHUNK)]],
                rows_v.at[b],
                gsem.at[b],
            )

        def writeback(j):
            b = j % _NBUF
            cps = []
            for r in range(_CROWS):
                cps.append(
                    pltpu.async_copy(
                        rows_v.at[b].at[pl.ds(r * SIG_LEN, SIG_LEN)],
                        out_hbm.at[bbase + j * _CROWS + r].at[
                            pl.ds(0, SIG_LEN), pl.ds(0, EMB_DIM)
                        ],
                        osem.at[b],
                    )
                )
            return cps

        gathers = [gather(j) for j in range(_NBUF - 1)]
        writebacks = [None] * _NCHUNK
        for j in range(_NCHUNK):
            gathers[j].wait()
            writebacks[j] = writeback(j)
            k = j + _NBUF - 1
            if k < _NCHUNK:
                if k >= _NBUF:
                    for cp in writebacks[k - _NBUF]:
                        cp.wait()
                gathers.append(gather(k))
        for j in range(_NCHUNK - _NBUF, _NCHUNK):
            for cp in writebacks[j]:
                cp.wait()

    return gather_kernel


_gather = _make_gather()


@jax.jit
def kernel(news_batch, table):
    idx2 = news_batch.astype(jnp.int32).reshape(NUM_IDX) * 2
    table_p = jnp.pad(table, ((0, 0), (0, _DPAD - EMB_DIM)))
    table_2m = table_p.reshape(2 * 1000000, EMB_DIM)
    out_p = _gather(idx2, table_2m)
    return out_p[:, :SIG_LEN, :EMB_DIM]
